# baseline (speedup 1.0000x reference)
import jax
import jax.numpy as jnp
from jax import lax
from jax.experimental import pallas as pl
from jax.experimental.pallas import tpu as pltpu

_ORDER = (4, 1, 7, 0, 2, 3, 5, 6, 8)


def _make_body(H, W, C, T):
    HW = H * W
    P = 128

    def body(x_ref, w1_ref, b1_ref, w2_ref, b2_ref, out_ref, pad_ref):
        pad_ref[:, :P] = jnp.zeros((C, P), jnp.bfloat16)
        pad_ref[:, P + HW:] = jnp.zeros((C, P), jnp.bfloat16)

        colid = lax.broadcasted_iota(jnp.int32, (C, HW), 1) % W
        ok_left = colid >= 1
        ok_right = colid <= W - 2
        zero = jnp.zeros((C, HW), jnp.bfloat16)

        def tap(k):
            oy, ox = k // 3 - 1, k % 3 - 1
            s = P + oy * W + ox
            v = pad_ref[:, s:s + HW]
            if ox == -1:
                v = jnp.where(ok_left, v, zero)
            elif ox == 1:
                v = jnp.where(ok_right, v, zero)
            return v

        def conv3x3(a_bf, wref, t, bref):
            pad_ref[:, P:P + HW] = a_bf
            y = bref[t]
            for j in range(4):
                k0, k1 = _ORDER[2 * j], _ORDER[2 * j + 1]
                op = jnp.concatenate([tap(k0), tap(k1)], axis=0)
                y = y + jnp.dot(wref[t, :, 2 * j * C:(2 * j + 2) * C], op,
                                preferred_element_type=jnp.float32)
            y = y + jnp.dot(wref[t, :, 8 * C:9 * C], tap(_ORDER[8]),
                            preferred_element_type=jnp.float32)
            return y

        def block(t, r):
            y1 = jnp.maximum(conv3x3(r.astype(jnp.bfloat16), w1_ref, t, b1_ref), 0.0)
            y2 = conv3x3(y1.astype(jnp.bfloat16), w2_ref, t, b2_ref)
            return r + y2

        out_ref[0] = lax.fori_loop(0, T, block, x_ref[0])

    return body


def kernel(x, w1, b1, w2, b2):
    N, C, H, W = x.shape
    T = w1.shape[0]
    HW = H * W
    P = 128

    def pack(w):
        wt = jnp.transpose(w, (0, 3, 1, 2))
        wt = wt[:, :, jnp.array(_ORDER), :]
        return wt.reshape(T, C, 9 * C).astype(jnp.bfloat16)

    w1m, w2m = pack(w1), pack(w2)
    b1m = jnp.transpose(b1, (0, 2, 1))
    b2m = jnp.transpose(b2, (0, 2, 1))

    xf = x.reshape(N, C, HW)
    out = pl.pallas_call(
        _make_body(H, W, C, T),
        out_shape=jax.ShapeDtypeStruct((N, C, HW), x.dtype),
        grid=(N,),
        in_specs=[
            pl.BlockSpec((1, C, HW), lambda n: (n, 0, 0)),
            pl.BlockSpec((T, C, 9 * C), lambda n: (0, 0, 0)),
            pl.BlockSpec((T, C, 1), lambda n: (0, 0, 0)),
            pl.BlockSpec((T, C, 9 * C), lambda n: (0, 0, 0)),
            pl.BlockSpec((T, C, 1), lambda n: (0, 0, 0)),
        ],
        out_specs=pl.BlockSpec((1, C, HW), lambda n: (n, 0, 0)),
        scratch_shapes=[
            pltpu.VMEM((C, HW + 2 * P), jnp.bfloat16),
        ],
        compiler_params=pltpu.CompilerParams(
            dimension_semantics=("parallel",)),
    )(xf, w1m, b1m, w2m, b2m)
    return out.reshape(N, C, H, W)


# speedup vs baseline: 1.2913x; 1.2913x over previous
import jax
import jax.numpy as jnp
from jax import lax
from jax.experimental import pallas as pl
from jax.experimental.pallas import tpu as pltpu


def _make_body(H, W, C, T):
    HW = H * W
    P = 128

    def body(x_ref, w1_ref, b1_ref, w2_ref, b2_ref, out_ref,
             pad0_ref, padl_ref, padr_ref, col_ref):
        for ref in (pad0_ref, padl_ref, padr_ref):
            ref[:, :P] = jnp.zeros((C, P), jnp.bfloat16)
            ref[:, P + HW:] = jnp.zeros((C, P), jnp.bfloat16)

        colid = lax.broadcasted_iota(jnp.int32, (C, HW), 1) % W
        not_last = colid != (W - 1)
        not_first = colid != 0
        zero = jnp.zeros((C, HW), jnp.bfloat16)

        def conv3x3(a_bf, wref, t, bref):
            pad0_ref[:, P:P + HW] = a_bf
            padl_ref[:, P:P + HW] = jnp.where(not_last, a_bf, zero)
            padr_ref[:, P:P + HW] = jnp.where(not_first, a_bf, zero)
            for k in range(9):
                oy, ox = k // 3 - 1, k % 3 - 1
                s = P + oy * W + ox
                src_ref = (padl_ref, pad0_ref, padr_ref)[ox + 1]
                col_ref[k * C:(k + 1) * C, :] = src_ref[:, s:s + HW]
            return jnp.dot(wref[t], col_ref[...],
                           preferred_element_type=jnp.float32) + bref[t]

        def block(t, carry):
            r = out_ref[0]
            y1 = jnp.maximum(conv3x3(r.astype(jnp.bfloat16), w1_ref, t, b1_ref), 0.0)
            y2 = conv3x3(y1.astype(jnp.bfloat16), w2_ref, t, b2_ref)
            out_ref[0] = r + y2
            return carry

        out_ref[0] = x_ref[0]
        lax.fori_loop(0, T, block, 0)

    return body


def kernel(x, w1, b1, w2, b2):
    N, C, H, W = x.shape
    T = w1.shape[0]
    HW = H * W
    P = 128

    def pack(w):
        wb = w.astype(jnp.bfloat16)
        return jnp.transpose(wb, (0, 3, 1, 2)).reshape(T, C, 9 * C)

    w1m, w2m = pack(w1), pack(w2)
    b1m = jnp.transpose(b1, (0, 2, 1))
    b2m = jnp.transpose(b2, (0, 2, 1))

    xf = x.reshape(N, C, HW)
    out = pl.pallas_call(
        _make_body(H, W, C, T),
        out_shape=jax.ShapeDtypeStruct((N, C, HW), x.dtype),
        grid=(N,),
        in_specs=[
            pl.BlockSpec((1, C, HW), lambda n: (n, 0, 0)),
            pl.BlockSpec((T, C, 9 * C), lambda n: (0, 0, 0)),
            pl.BlockSpec((T, C, 1), lambda n: (0, 0, 0)),
            pl.BlockSpec((T, C, 9 * C), lambda n: (0, 0, 0)),
            pl.BlockSpec((T, C, 1), lambda n: (0, 0, 0)),
        ],
        out_specs=pl.BlockSpec((1, C, HW), lambda n: (n, 0, 0)),
        scratch_shapes=[
            pltpu.VMEM((C, HW + 2 * P), jnp.bfloat16),
            pltpu.VMEM((C, HW + 2 * P), jnp.bfloat16),
            pltpu.VMEM((C, HW + 2 * P), jnp.bfloat16),
            pltpu.VMEM((9 * C, HW), jnp.bfloat16),
        ],
        compiler_params=pltpu.CompilerParams(
            dimension_semantics=("parallel",)),
    )(xf, w1m, b1m, w2m, b2m)
    return out.reshape(N, C, H, W)
